# baseline (speedup 1.0000x reference)
import jax
import jax.numpy as jnp
from jax import lax
from jax.experimental import pallas as pl
from jax.experimental.pallas import tpu as pltpu
from jax.experimental.pallas import tpu_sc as plsc

NC = 2
NS = 16
NW = NC * NS
L = 16

B = 4
M = 8192
N = B * M
PPW = N // NW
CHUNK = 16
NCHUNK = PPW // CHUNK

C0 = 32
S0, S1 = 64, 32
CAP0 = float(S0 - 1.01)
CAP1 = float(S1 - 1.01)
LO = 0.01


def _vol_setup(px, py, pz, scale, cap, bbase, s):
    def axis(p):
        t = jnp.clip(p * scale, LO, cap)
        i1 = t.astype(jnp.int32)
        f1 = i1.astype(jnp.float32)
        i2 = i1 + jnp.where(t != f1, 1, 0)
        f2 = i2.astype(jnp.float32)
        return i1, i2, t - f1, f2 - t

    x1, x2, wx, wx2 = axis(px)
    y1, y2, wy, wy2 = axis(py)
    z1, z2, wz, wz2 = axis(pz)

    s2 = s * s
    ax = (bbase + x1 * s2, bbase + x2 * s2)
    ay = (y1 * s, y2 * s)
    az = (z1, z2)
    wxs = (wx2, wx)
    wys = (wy2, wy)
    wzs = (wz2, wz)

    rows = []
    weights = []
    for dx in range(2):
        for dy in range(2):
            wxy = wxs[dx] * wys[dy]
            axy = ax[dx] + ay[dy]
            for dz in range(2):
                rows.append(axy + az[dz])
                weights.append(wxy * wzs[dz])
    return rows, weights


def _body(f0, f1, xs, ys, zs, out,
          xv, yv, zv, idx0, idx1, rows0, rows1, outv, sem0, sem1):
    cid = lax.axis_index("c")
    sid = lax.axis_index("s")
    wid = sid * NC + cid
    base = wid * PPW

    pltpu.sync_copy(xs.at[pl.ds(base, PPW)], xv)
    pltpu.sync_copy(ys.at[pl.ds(base, PPW)], yv)
    pltpu.sync_copy(zs.at[pl.ds(base, PPW)], zv)

    lane = lax.iota(jnp.int32, L)
    rowidx = [lane + k * L for k in range(8)]

    @pl.loop(0, NCHUNK)
    def _chunk(ci):
        p0 = ci * CHUNK
        px = xv[pl.ds(p0, CHUNK)]
        py = yv[pl.ds(p0, CHUNK)]
        pz = zv[pl.ds(p0, CHUNK)]
        g = base + p0 + lane
        b = lax.shift_right_logical(g, 13)

        r0, w0 = _vol_setup(px, py, pz, float(S0), CAP0,
                            lax.shift_left(b, 18), S0)
        r1, w1 = _vol_setup(px, py, pz, float(S1), CAP1,
                            lax.shift_left(b, 15), S1)
        for k in range(8):
            idx0[pl.ds(k * L, L)] = r0[k]
            idx1[pl.ds(k * L, L)] = r1[k]

        cp0 = pltpu.async_copy(f0.at[idx0], rows0, sem0)
        cp1 = pltpu.async_copy(f1.at[idx1], rows1, sem1)

        pidx = lane + p0

        def vol_compute(rows, w, choff):
            for ch in range(C0):
                chv = jnp.full((L,), ch, jnp.int32)
                acc = plsc.load_gather(rows, [rowidx[0], chv]) * w[0]
                for k in range(1, 8):
                    acc = acc + plsc.load_gather(rows, [rowidx[k], chv]) * w[k]
                plsc.store_scatter(
                    outv, [pidx, jnp.full((L,), ch + choff, jnp.int32)], acc)

        cp0.wait()
        vol_compute(rows0, w0, 0)
        cp1.wait()
        vol_compute(rows1, w1, C0)

    pltpu.sync_copy(outv, out.at[pl.ds(base, PPW)])


@jax.jit
def kernel(feat0, feat1, mesh_coords):
    f0 = feat0.reshape(B * S0 * S0 * S0, C0)
    f1 = feat1.reshape(B * S1 * S1 * S1, C0)
    c = mesh_coords.reshape(N, 3)
    xs = c[:, 0]
    ys = c[:, 1]
    zs = c[:, 2]

    mesh = plsc.VectorSubcoreMesh(
        core_axis_name="c", subcore_axis_name="s",
        num_cores=NC, num_subcores=NS)
    run = pl.kernel(
        _body,
        out_type=jax.ShapeDtypeStruct((N, 2 * C0), jnp.float32),
        mesh=mesh,
        scratch_types=[
            pltpu.VMEM((PPW,), jnp.float32),
            pltpu.VMEM((PPW,), jnp.float32),
            pltpu.VMEM((PPW,), jnp.float32),
            pltpu.VMEM((8 * L,), jnp.int32),
            pltpu.VMEM((8 * L,), jnp.int32),
            pltpu.VMEM((8 * L, C0), jnp.float32),
            pltpu.VMEM((8 * L, C0), jnp.float32),
            pltpu.VMEM((PPW, 2 * C0), jnp.float32),
            pltpu.SemaphoreType.DMA,
            pltpu.SemaphoreType.DMA,
        ],
        compiler_params=pltpu.CompilerParams(
            needs_layout_passes=False, use_tc_tiling_on_sc=False),
    )
    out = run(f0, f1, xs, ys, zs)
    return out.reshape(B, M, 2 * C0)
